# baseline (speedup 1.0000x reference)
import functools

import jax
import jax.numpy as jnp
from jax import lax
from jax.experimental import pallas as pl
from jax.experimental.pallas import tpu as pltpu
from jax.experimental.pallas import tpu_sc as plsc

_NUM_STREETS = 4
_EMBED_DIM = 128
_MAX_SEQ_LEN = 128
_VOCAB = _NUM_STREETS + 3
_PAD_TOKEN = _NUM_STREETS + 2
_LANES = 16
_NREP = 512
_HEAD = 64


def _sc_call(street_flat, table_rep, padtail, b, lcur, idx_dtype):
    info = plsc.get_sparse_core_info()
    nc, ns = info.num_cores, info.num_subcores
    nw = nc * ns
    rows_w = b // nw
    bt = b * _MAX_SEQ_LEN
    nvec = _MAX_SEQ_LEN // _LANES
    hvec = _HEAD // _LANES
    nfull = lcur // _LANES
    rem = lcur - nfull * _LANES
    hcnt = -(-lcur // 8) * 8
    tail = _MAX_SEQ_LEN - hcnt
    tvec = -(-tail // _LANES)

    mesh = plsc.VectorSubcoreMesh(core_axis_name="c", subcore_axis_name="s")
    nb = 4
    cr = 2
    ctoks = cr * _HEAD
    nchunks = rows_w // cr
    toks_w = rows_w * _MAX_SEQ_LEN

    @functools.partial(
        pl.kernel,
        mesh=mesh,
        out_type=[
            jax.ShapeDtypeStruct((bt,), idx_dtype),
            jax.ShapeDtypeStruct((bt, _EMBED_DIM), jnp.float32),
        ],
        scratch_types=[
            pltpu.VMEM((rows_w * lcur + _LANES,), jnp.int32),
            pltpu.VMEM((toks_w,), jnp.int32),
            pltpu.VMEM((nb * ctoks,), jnp.int32),
            pltpu.VMEM((nb, ctoks, _EMBED_DIM), jnp.float32),
        ] + [pltpu.SemaphoreType.DMA] * (2 * nb),
    )
    def kern(street_hbm, table_hbm, padtail_hbm, idxs_hbm, emb_hbm,
             sidx_v, stage_v, gidx_v, rows_v, *sems):
        gsem = sems[:nb]
        wsem = sems[nb:]
        wid = lax.axis_index("s") * nc + lax.axis_index("c")
        row0 = wid * rows_w

        pltpu.sync_copy(street_hbm.at[pl.ds(row0 * lcur, rows_w * lcur)],
                        sidx_v.at[pl.ds(0, rows_w * lcur)])

        lane = lax.iota(jnp.int32, _LANES)
        pad_v = jnp.full((_LANES,), _PAD_TOKEN, jnp.int32)
        shifts = [
            jnp.bitwise_and(lane + (p * _LANES) + wid * _LANES, _NREP - 1)
            * _VOCAB
            for p in range(cr * nvec)
        ]


        def build(buf, g):
            for r in range(cr):
                base = (g * cr + r) * lcur
                sbase = (g * cr + r) * _MAX_SEQ_LEN
                for j in range(nvec):
                    if j < nfull:
                        v = sidx_v[pl.ds(base + j * _LANES, _LANES)]
                    elif j == nfull and rem:
                        raw = sidx_v[pl.ds(base + nfull * _LANES, _LANES)]
                        v = jnp.where(lane < rem, raw, pad_v)
                    else:
                        v = pad_v
                    stage_v[pl.ds(sbase + j * _LANES, _LANES)] = v
                    if j < hvec:
                        gidx_v[pl.ds(buf * ctoks + r * _HEAD + j * _LANES,
                                     _LANES)] = v + shifts[r * nvec + j]

        def gather_pair(buf, r):
            src = table_hbm.at[gidx_v.at[pl.ds(buf * ctoks + r * _HEAD,
                                               hcnt)]]
            dst = rows_v.at[buf, pl.ds(r * _HEAD, hcnt)]
            return src, dst

        def start_gather(buf):
            for r in range(cr):
                src, dst = gather_pair(buf, r)
                pltpu.async_copy(src, dst, gsem[buf])

        def wait_gather(buf):
            for r in range(cr):
                src, dst = gather_pair(buf, r)
                pltpu.make_async_copy(src, dst, gsem[buf]).wait()

        def head_pair(buf, g, r):
            src = rows_v.at[buf, pl.ds(r * _HEAD, hcnt)]
            dst = emb_hbm.at[pl.ds((row0 + g * cr + r) * _MAX_SEQ_LEN,
                                   hcnt)]
            return src, dst

        def tail_dst(g, r):
            return emb_hbm.at[
                pl.ds((row0 + g * cr + r) * _MAX_SEQ_LEN + hcnt, tail)]

        def start_write(buf, g):
            for r in range(cr):
                src, dst = head_pair(buf, g, r)
                pltpu.async_copy(src, dst, wsem[buf])
                pltpu.async_copy(padtail_hbm, tail_dst(g, r), wsem[buf])

        def wait_write(buf, g):
            for r in range(cr):
                src, dst = head_pair(buf, g, r)
                pltpu.make_async_copy(src, dst, wsem[buf]).wait()
                pltpu.make_async_copy(padtail_hbm, tail_dst(g, r),
                                      wsem[buf]).wait()

        for bf in range(nb):
            build(bf, bf)
            start_gather(bf)

        def loop_body(i, carry):
            g0 = i * nb
            for bf in range(nb):
                wait_gather(bf)
                start_write(bf, g0 + bf)
            for bf in range(nb):
                build(bf, g0 + nb + bf)
                wait_write(bf, g0 + bf)
                start_gather(bf)
            return carry

        lax.fori_loop(0, nchunks // nb - 1, loop_body, 0)
        g0 = nchunks - nb
        for bf in range(nb):
            wait_gather(bf)
            start_write(bf, g0 + bf)
        for bf in range(nb):
            wait_write(bf, g0 + bf)

        pltpu.sync_copy(stage_v, idxs_hbm.at[pl.ds(row0 * _MAX_SEQ_LEN,
                                                   toks_w)])

    return kern(street_flat, table_rep, padtail)


def kernel(street_idxs, table):
    b, lcur = street_idxs.shape
    street_flat = street_idxs.reshape(-1)
    table_rep = jnp.tile(table, (_NREP, 1))
    hcnt = -(-lcur // 8) * 8
    padtail = jnp.tile(table[_PAD_TOKEN:_PAD_TOKEN + 1],
                       (_MAX_SEQ_LEN - hcnt, 1))
    idxs_f, emb_f = _sc_call(street_flat, table_rep, padtail, b, lcur,
                             street_idxs.dtype)
    return (idxs_f.reshape(b, _MAX_SEQ_LEN),
            emb_f.reshape(b, _MAX_SEQ_LEN, _EMBED_DIM))


# speedup vs baseline: 24.7834x; 24.7834x over previous
import functools

import jax
import jax.numpy as jnp
from jax import lax
from jax.experimental import pallas as pl
from jax.experimental.pallas import tpu as pltpu
from jax.experimental.pallas import tpu_sc as plsc

_NUM_STREETS = 4
_EMBED_DIM = 128
_MAX_SEQ_LEN = 128
_VOCAB = _NUM_STREETS + 3
_PAD_TOKEN = _NUM_STREETS + 2
_LANES = 16
_NREP = 512
_HEAD = 64


def _sc_call(street_flat, table_rep, b, lcur, idx_dtype):
    info = plsc.get_sparse_core_info()
    nc, ns = info.num_cores, info.num_subcores
    nw = nc * ns
    rows_w = b // nw
    bt = b * _MAX_SEQ_LEN
    nvec = _MAX_SEQ_LEN // _LANES
    hvec = _HEAD // _LANES
    nfull = lcur // _LANES
    rem = lcur - nfull * _LANES
    hcnt = -(-lcur // 8) * 8
    tail = _MAX_SEQ_LEN - hcnt
    tvec = -(-tail // _LANES)

    mesh = plsc.VectorSubcoreMesh(core_axis_name="c", subcore_axis_name="s")
    nb = 4
    cr = 2
    ctoks = cr * _HEAD
    nchunks = rows_w // cr
    toks_w = rows_w * _MAX_SEQ_LEN

    @functools.partial(
        pl.kernel,
        mesh=mesh,
        out_type=[
            jax.ShapeDtypeStruct((bt,), idx_dtype),
            jax.ShapeDtypeStruct((bt, _EMBED_DIM), jnp.float32),
        ],
        scratch_types=[
            pltpu.VMEM((rows_w * lcur + _LANES,), jnp.int32),
            pltpu.VMEM((toks_w,), jnp.int32),
            pltpu.VMEM((nb * ctoks,), jnp.int32),
            pltpu.VMEM((nb, ctoks, _EMBED_DIM), jnp.float32),
            pltpu.VMEM((tail, _EMBED_DIM), jnp.float32),
        ] + [pltpu.SemaphoreType.DMA] * (2 * nb),
    )
    def kern(street_hbm, table_hbm, idxs_hbm, emb_hbm,
             sidx_v, stage_v, gidx_v, rows_v, ctail_v, *sems):
        gsem = sems[:nb]
        wsem = sems[nb:]
        wid = lax.axis_index("s") * nc + lax.axis_index("c")
        row0 = wid * rows_w

        pltpu.sync_copy(street_hbm.at[pl.ds(row0 * lcur, rows_w * lcur)],
                        sidx_v.at[pl.ds(0, rows_w * lcur)])

        lane = lax.iota(jnp.int32, _LANES)
        pad_v = jnp.full((_LANES,), _PAD_TOKEN, jnp.int32)
        shifts = [
            jnp.bitwise_and(lane + (p * _LANES) + wid * _LANES, _NREP - 1)
            * _VOCAB
            for p in range(cr * nvec)
        ]


        for p in range(tvec):
            gidx_v[pl.ds(p * _LANES, _LANES)] = shifts[p] + _PAD_TOKEN
        pltpu.async_copy(table_hbm.at[gidx_v.at[pl.ds(0, tail)]], ctail_v,
                         gsem[0])
        pltpu.make_async_copy(table_hbm.at[gidx_v.at[pl.ds(0, tail)]],
                              ctail_v, gsem[0]).wait()

        def build(buf, g):
            for r in range(cr):
                base = (g * cr + r) * lcur
                sbase = (g * cr + r) * _MAX_SEQ_LEN
                for j in range(nvec):
                    if j < nfull:
                        v = sidx_v[pl.ds(base + j * _LANES, _LANES)]
                    elif j == nfull and rem:
                        raw = sidx_v[pl.ds(base + nfull * _LANES, _LANES)]
                        v = jnp.where(lane < rem, raw, pad_v)
                    else:
                        v = pad_v
                    stage_v[pl.ds(sbase + j * _LANES, _LANES)] = v
                    if j < hvec:
                        gidx_v[pl.ds(buf * ctoks + r * _HEAD + j * _LANES,
                                     _LANES)] = v + shifts[r * nvec + j]

        def gather_pair(buf, r):
            src = table_hbm.at[gidx_v.at[pl.ds(buf * ctoks + r * _HEAD,
                                               hcnt)]]
            dst = rows_v.at[buf, pl.ds(r * _HEAD, hcnt)]
            return src, dst

        def start_gather(buf):
            for r in range(cr):
                src, dst = gather_pair(buf, r)
                pltpu.async_copy(src, dst, gsem[buf])

        def wait_gather(buf):
            for r in range(cr):
                src, dst = gather_pair(buf, r)
                pltpu.make_async_copy(src, dst, gsem[buf]).wait()

        def head_pair(buf, g, r):
            src = rows_v.at[buf, pl.ds(r * _HEAD, hcnt)]
            dst = emb_hbm.at[pl.ds((row0 + g * cr + r) * _MAX_SEQ_LEN,
                                   hcnt)]
            return src, dst

        def tail_dst(g, r):
            return emb_hbm.at[
                pl.ds((row0 + g * cr + r) * _MAX_SEQ_LEN + hcnt, tail)]

        def start_write(buf, g):
            for r in range(cr):
                src, dst = head_pair(buf, g, r)
                pltpu.async_copy(src, dst, wsem[buf])
                pltpu.async_copy(ctail_v, tail_dst(g, r), wsem[buf])

        def wait_write(buf, g):
            for r in range(cr):
                src, dst = head_pair(buf, g, r)
                pltpu.make_async_copy(src, dst, wsem[buf]).wait()
                pltpu.make_async_copy(ctail_v, tail_dst(g, r),
                                      wsem[buf]).wait()

        for bf in range(nb):
            build(bf, bf)
            start_gather(bf)

        def loop_body(i, carry):
            g0 = i * nb
            for bf in range(nb):
                wait_gather(bf)
                start_write(bf, g0 + bf)
            for bf in range(nb):
                build(bf, g0 + nb + bf)
                wait_write(bf, g0 + bf)
                start_gather(bf)
            return carry

        lax.fori_loop(0, nchunks // nb - 1, loop_body, 0)
        g0 = nchunks - nb
        for bf in range(nb):
            wait_gather(bf)
            start_write(bf, g0 + bf)
        for bf in range(nb):
            wait_write(bf, g0 + bf)

        pltpu.sync_copy(stage_v, idxs_hbm.at[pl.ds(row0 * _MAX_SEQ_LEN,
                                                   toks_w)])

    return kern(street_flat, table_rep)


def kernel(street_idxs, table):
    b, lcur = street_idxs.shape
    street_flat = street_idxs.reshape(-1)
    table_rep = jnp.tile(table, (_NREP, 1))
    idxs_f, emb_f = _sc_call(street_flat, table_rep, b, lcur,
                             street_idxs.dtype)
    return (idxs_f.reshape(b, _MAX_SEQ_LEN),
            emb_f.reshape(b, _MAX_SEQ_LEN, _EMBED_DIM))
